# baseline (speedup 1.0000x reference)
import functools

import jax
import jax.numpy as jnp
from jax.experimental import pallas as pl
from jax.experimental.pallas import tpu as pltpu

N = 10000
D = 512
BI = 80


def _body(x_ref, w_ref, bias_ref, adj_ref, out_ref, sup_ref):
    @pl.when(pl.program_id(0) == 0)
    def _init():
        sup_ref[...] = jnp.dot(
            x_ref[...].astype(jnp.bfloat16),
            w_ref[...],
            preferred_element_type=jnp.float32,
        ).astype(jnp.bfloat16)

    out_ref[...] = (
        jnp.dot(
            adj_ref[...].astype(jnp.bfloat16),
            sup_ref[...],
            preferred_element_type=jnp.float32,
        )
        + bias_ref[...]
    )


@functools.partial(jax.jit, static_argnames=())
def kernel(adj, input, weight, bias):
    w_bf = weight.astype(jnp.bfloat16)
    bias2d = bias.reshape(1, D)

    out = pl.pallas_call(
        _body,
        grid=(N // BI,),
        in_specs=[
            pl.BlockSpec((N, D), lambda i: (0, 0)),
            pl.BlockSpec((D, D), lambda i: (0, 0)),
            pl.BlockSpec((1, D), lambda i: (0, 0)),
            pl.BlockSpec((BI, N), lambda i: (i, 0)),
        ],
        out_specs=pl.BlockSpec((BI, D), lambda i: (i, 0)),
        out_shape=jax.ShapeDtypeStruct((N, D), jnp.float32),
        scratch_shapes=[pltpu.VMEM((N, D), jnp.bfloat16)],
        compiler_params=pltpu.CompilerParams(
            dimension_semantics=("arbitrary",),
        ),
    )(input, w_bf, bias2d, adj)

    return out


# speedup vs baseline: 1.6865x; 1.6865x over previous
import functools

import jax
import jax.numpy as jnp
from jax.experimental import pallas as pl
from jax.experimental.pallas import tpu as pltpu

N = 10000
D = 512
BI1 = 1000
BI2 = 400


def _mm1_body(x_ref, w_ref, out_ref):
    out_ref[...] = jnp.dot(
        x_ref[...].astype(jnp.bfloat16),
        w_ref[...],
        preferred_element_type=jnp.float32,
    ).astype(jnp.bfloat16)


def _mm2_body(adj_ref, sup_ref, bias_ref, out_ref):
    a = adj_ref[...].astype(jnp.bfloat16)
    out_ref[...] = (
        jnp.dot(a, sup_ref[...], preferred_element_type=jnp.float32)
        + bias_ref[...]
    )


@functools.partial(jax.jit, static_argnames=())
def kernel(adj, input, weight, bias):
    w_bf = weight.astype(jnp.bfloat16)
    bias2d = bias.reshape(1, D)

    support = pl.pallas_call(
        _mm1_body,
        grid=(N // BI1,),
        in_specs=[
            pl.BlockSpec((BI1, D), lambda i: (i, 0)),
            pl.BlockSpec((D, D), lambda i: (0, 0)),
        ],
        out_specs=pl.BlockSpec((BI1, D), lambda i: (i, 0)),
        out_shape=jax.ShapeDtypeStruct((N, D), jnp.bfloat16),
        compiler_params=pltpu.CompilerParams(
            dimension_semantics=("arbitrary",),
        ),
    )(input, w_bf)

    out = pl.pallas_call(
        _mm2_body,
        grid=(N // BI2,),
        in_specs=[
            pl.BlockSpec((BI2, N), lambda i: (i, 0)),
            pl.BlockSpec((N, D), lambda i: (0, 0)),
            pl.BlockSpec((1, D), lambda i: (0, 0)),
        ],
        out_specs=pl.BlockSpec((BI2, D), lambda i: (i, 0)),
        out_shape=jax.ShapeDtypeStruct((N, D), jnp.float32),
        compiler_params=pltpu.CompilerParams(
            dimension_semantics=("arbitrary",),
        ),
    )(adj, support, bias2d)

    return out
